# baseline (speedup 1.0000x reference)
import functools

import jax
import jax.numpy as jnp
from jax import lax
from jax.experimental import pallas as pl
from jax.experimental.pallas import tpu as pltpu
from jax.experimental.pallas import tpu_sc as plsc

D = 128
V = 98
VP = 104
S = 32
L = 16
NC = 2
NS = 16
NW = NC * NS
B = 16384
TOK = B * S
R = TOK // NW // 128


def _mesh():
    return plsc.VectorSubcoreMesh(
        core_axis_name="c", subcore_axis_name="s", num_cores=NC, num_subcores=NS
    )


def _pos_encoding():
    positions = jnp.arange(S, dtype=jnp.float32)
    power_values = jnp.power(
        1000.0, 2.0 * jnp.arange(0, D, 2, dtype=jnp.float32) / D
    )
    angle = positions[:, None] / power_values[None, :]
    pe = jnp.zeros((S, D), dtype=jnp.float32)
    pe = pe.at[:, 0::2].set(jnp.sin(angle))
    pe = pe.at[:, 1::2].set(jnp.cos(angle))
    return pe


def _embed(x2, W, pe):
    @functools.partial(
        pl.kernel,
        out_type=jax.ShapeDtypeStruct((TOK, D), jnp.float32),
        mesh=_mesh(),
        scratch_types=[
            pltpu.VMEM_SHARED((S * VP, D), jnp.float32),
            pltpu.VMEM((VP, D), jnp.float32),
            pltpu.VMEM((S, D), jnp.float32),
            pltpu.VMEM((R, 128), jnp.int32),
            pltpu.VMEM((8, 64, D), jnp.float32),
            pltpu.SemaphoreType.DMA,
            pltpu.SemaphoreType.DMA,
            pltpu.SemaphoreType.DMA,
            pltpu.SemaphoreType.DMA,
            pltpu.SemaphoreType.DMA,
        ],
    )
    def k(x_hbm, w_hbm, pe_hbm, out_hbm, tsh, tb, pev, xi, data,
          gsem, wsem, xsem, lsem, tsem):
        cid = lax.axis_index("c")
        sid = lax.axis_index("s")
        wid = sid * NC + cid

        cp_x = pltpu.async_copy(x_hbm.at[pl.ds(wid * R, R)], xi, xsem)
        cp_pe = pltpu.async_copy(pe_hbm, pev, lsem)
        cp_w = pltpu.async_copy(w_hbm, tb.at[pl.ds(0, V)], lsem)
        cp_pe.wait()
        cp_w.wait()
        s0 = sid
        s1 = sid + NS

        @pl.loop(0, V)
        def _(r):
            for c in range(D // L):
                sl = pl.ds(c * L, L)
                tb[r, sl] = tb[r, sl] + pev[s0, sl]

        pltpu.sync_copy(tb, tsh.at[pl.ds(s0 * VP, VP)])
        cp_w = pltpu.async_copy(w_hbm, tb.at[pl.ds(0, V)], lsem)
        cp_w.wait()

        @pl.loop(0, V)
        def _(r):
            for c in range(D // L):
                sl = pl.ds(c * L, L)
                tb[r, sl] = tb[r, sl] + pev[s1, sl]

        pltpu.async_copy(tb, tsh.at[pl.ds(s1 * VP, VP)], tsem)

        cp_x.wait()
        iota = lax.broadcasted_iota(jnp.int32, (L,), 0)
        e0 = iota * VP
        e1 = (iota + L) * VP

        @pl.loop(0, R)
        def _(p):
            for c in range(D // L):
                sl = pl.ds(c * L, L)
                xi[p, sl] = xi[p, sl] + (e0 if c % 2 == 0 else e1)

        pltpu.make_async_copy(tb, tsh.at[pl.ds(s1 * VP, VP)], tsem).wait()

        plsc.subcore_barrier()

        outbase = wid * (R * 128)
        NCH = 2 * R

        def g_start(row, half, b):
            idx = xi.at[row, pl.ds(half * 64, 64)]
            pltpu.async_copy(tsh.at[idx], data.at[b], gsem)

        def g_wait(row, half, b):
            idx = xi.at[row, pl.ds(half * 64, 64)]
            pltpu.make_async_copy(tsh.at[idx], data.at[b], gsem).wait()

        def w_start(k, b):
            pltpu.async_copy(
                data.at[b], out_hbm.at[pl.ds(outbase + k * 64, 64)], wsem
            )

        def w_wait(k, b):
            pltpu.make_async_copy(
                data.at[b], out_hbm.at[pl.ds(outbase + k * 64, 64)], wsem
            ).wait()

        for j in range(4):
            g_start(j // 2, j % 2, j)
        for k in range(4):
            g_wait(k // 2, k % 2, k)
            w_start(k, k)
            g_start((k + 4) // 2, k % 2, k + 4)

        @pl.loop(4, NCH - 4, step=8)
        def _(g):
            row0 = g // 2
            for b in range(8):
                k = g + b
                bb = (4 + b) % 8
                g_wait(row0 + b // 2, b % 2, bb)
                w_start(k, bb)
                w_wait(k - 4, b % 8)
                g_start(row0 + (b + 4) // 2, b % 2, b % 8)

        for j in range(4):
            k = NCH - 4 + j
            b = k % 8
            g_wait(k // 2, k % 2, b)
            w_start(k, b)
            w_wait(k - 4, (b + 4) % 8)
        for j in range(4):
            w_wait(NCH - 4 + j, (NCH - 4 + j) % 8)

    return k(x2, W, pe)


def kernel(x, start_token, end_token, W):
    del start_token, end_token
    x2 = x.reshape(TOK // 128, 128)
    out = _embed(x2, W, _pos_encoding())
    return out.reshape(B, S, D)


# speedup vs baseline: 1.0000x; 1.0000x over previous
import functools

import jax
import jax.numpy as jnp
from jax import lax
from jax.experimental import pallas as pl
from jax.experimental.pallas import tpu as pltpu
from jax.experimental.pallas import tpu_sc as plsc

D = 128
V = 98
VP = 104
S = 32
L = 16
NC = 2
NS = 16
NW = NC * NS
B = 16384
TOK = B * S
R = TOK // NW // 128


def _mesh():
    return plsc.VectorSubcoreMesh(
        core_axis_name="c", subcore_axis_name="s", num_cores=NC, num_subcores=NS
    )


def _pos_encoding():
    positions = jnp.arange(S, dtype=jnp.float32)
    power_values = jnp.power(
        1000.0, 2.0 * jnp.arange(0, D, 2, dtype=jnp.float32) / D
    )
    angle = positions[:, None] / power_values[None, :]
    pe = jnp.zeros((S, D), dtype=jnp.float32)
    pe = pe.at[:, 0::2].set(jnp.sin(angle))
    pe = pe.at[:, 1::2].set(jnp.cos(angle))
    return pe


def _embed(x2, W, pe):
    @functools.partial(
        pl.kernel,
        out_type=jax.ShapeDtypeStruct((TOK, D), jnp.float32),
        mesh=_mesh(),
        scratch_types=[
            pltpu.VMEM_SHARED((S * VP, D), jnp.float32),
            pltpu.VMEM((VP, D), jnp.float32),
            pltpu.VMEM((S, D), jnp.float32),
            pltpu.VMEM((R, 128), jnp.int32),
            pltpu.VMEM((8, 64, D), jnp.float32),
            pltpu.SemaphoreType.DMA,
            pltpu.SemaphoreType.DMA,
            pltpu.SemaphoreType.DMA,
            pltpu.SemaphoreType.DMA,
            pltpu.SemaphoreType.DMA,
        ],
    )
    def k(x_hbm, w_hbm, pe_hbm, out_hbm, tsh, tb, pev, xi, data,
          gsem, wsem, xsem, lsem, tsem):
        cid = lax.axis_index("c")
        sid = lax.axis_index("s")
        wid = sid * NC + cid

        cp_x = pltpu.async_copy(x_hbm.at[pl.ds(wid * R, R)], xi, xsem)
        cp_pe = pltpu.async_copy(pe_hbm, pev, lsem)
        cp_w = pltpu.async_copy(w_hbm, tb.at[pl.ds(0, V)], lsem)
        cp_pe.wait()
        cp_w.wait()
        s0 = sid
        s1 = sid + NS

        @pl.loop(0, V)
        def _(r):
            for c in range(D // L):
                sl = pl.ds(c * L, L)
                tb[r, sl] = tb[r, sl] + pev[s0, sl]

        pltpu.sync_copy(tb, tsh.at[pl.ds(s0 * VP, VP)])
        cp_w = pltpu.async_copy(w_hbm, tb.at[pl.ds(0, V)], lsem)
        cp_w.wait()

        @pl.loop(0, V)
        def _(r):
            for c in range(D // L):
                sl = pl.ds(c * L, L)
                tb[r, sl] = tb[r, sl] + pev[s1, sl]

        pltpu.async_copy(tb, tsh.at[pl.ds(s1 * VP, VP)], tsem)

        cp_x.wait()
        iota = lax.broadcasted_iota(jnp.int32, (L,), 0)
        e0 = iota * VP
        e1 = (iota + L) * VP

        @pl.loop(0, R)
        def _(p):
            for c in range(D // L):
                sl = pl.ds(c * L, L)
                xi[p, sl] = xi[p, sl] + (e0 if c % 2 == 0 else e1)

        pltpu.make_async_copy(tb, tsh.at[pl.ds(s1 * VP, VP)], tsem).wait()

        plsc.subcore_barrier()

        outbase = wid * (R * 128)
        NCH = 2 * R

        def g_start(row, half, b):
            idx = xi.at[row, pl.ds(half * 64, 64)]
            pltpu.async_copy(tsh.at[idx], data.at[b], gsem)

        def g_wait(row, half, b):
            idx = xi.at[row, pl.ds(half * 64, 64)]
            pltpu.make_async_copy(tsh.at[idx], data.at[b], gsem).wait()

        def w_start(k, b):
            pltpu.async_copy(
                data.at[b], out_hbm.at[pl.ds(outbase + k * 64, 64)], wsem
            )

        def w_wait(k, b):
            pltpu.make_async_copy(
                data.at[b], out_hbm.at[pl.ds(outbase + k * 64, 64)], wsem
            ).wait()

        for j in range(4):
            g_start(j // 2, j % 2, j)
        for k in range(4):
            g_wait(k // 2, k % 2, k)
            w_start(k, k)
            g_start((k + 4) // 2, k % 2, k + 4)

        @pl.loop(4, NCH - 4, step=8)
        def _(g):
            row0 = g // 2
            for b in range(8):
                k = g + b
                bb = (4 + b) % 8
                w_wait(k - 4, b % 8)
                g_start(row0 + (b + 4) // 2, b % 2, b % 8)
                g_wait(row0 + b // 2, b % 2, bb)
                w_start(k, bb)

        for j in range(4):
            k = NCH - 4 + j
            b = k % 8
            g_wait(k // 2, k % 2, b)
            w_start(k, b)
            w_wait(k - 4, (b + 4) % 8)
        for j in range(4):
            w_wait(NCH - 4 + j, (NCH - 4 + j) % 8)

    return k(x2, W, pe)


def kernel(x, start_token, end_token, W):
    del start_token, end_token
    x2 = x.reshape(TOK // 128, 128)
    out = _embed(x2, W, _pos_encoding())
    return out.reshape(B, S, D)
